# baseline (speedup 1.0000x reference)
import functools

import jax
import jax.numpy as jnp
from jax.experimental import pallas as pl

HIDDEN_DIM = 64
VOCAB_SIZE = 64
MEMORY_SLOTS = 8
NEG = -1e30

INTERPRET = False


def _fused_kernel(L_real, BB, seq_ref, embed_ref, W1_ref, b1_ref, W2_ref,
                  b2_ref, gamma_ref, beta_ref, wg_ref, bg_ref, Wq_ref,
                  bq_ref, Wo_ref, bo_ref, out_ref):
    H = HIDDEN_DIM
    V = VOCAB_SIZE
    f32 = jnp.float32

    E = embed_ref[...]
    p1 = jnp.dot(E, W1_ref[...], preferred_element_type=f32)
    ff1 = jnp.maximum(p1 + b1_ref[...], 0.0)
    ff = jnp.dot(ff1, W2_ref[...], preferred_element_type=f32) + b2_ref[...]
    x = E + ff
    mu = jnp.mean(x, axis=-1, keepdims=True)
    var = jnp.mean((x - mu) ** 2, axis=-1, keepdims=True)
    HT = (x - mu) / jnp.sqrt(var + 1e-5) * gamma_ref[...] + beta_ref[...]

    st = jnp.dot(HT, wg_ref[...], preferred_element_type=f32) + bg_ref[0, 0]
    q_all = jnp.dot(HT, Wq_ref[...], preferred_element_type=f32) + bq_ref[...]
    LT = jax.lax.dot_general(HT, q_all, (((1,), (1,)), ((), ())),
                             preferred_element_type=f32) * (H ** -0.5)
    OT = jnp.dot(HT, Wo_ref[...], preferred_element_type=f32)
    G = (st > st.reshape(1, V)).astype(f32)

    seq = seq_ref[...]
    seq_t = seq.T
    tok = jax.lax.broadcasted_iota(jnp.int32, (V, L_real, BB), 0)
    oh = (seq_t[None, :, :] == tok).astype(f32)
    ones_l = jnp.ones((V, 1, L_real), dtype=f32)
    counts = jax.lax.dot_general(
        ones_l, oh, (((2,), (1,)), ((0,), (0,))),
        preferred_element_type=f32).reshape(V, BB).T

    S = jnp.dot(counts, G, preferred_element_type=f32)
    taken = jnp.minimum(jnp.maximum(8.0 - S, 0.0), counts)

    q_tok = seq[:, L_real - 1][:, None]
    qoh = (q_tok == jax.lax.broadcasted_iota(
        jnp.int32, (BB, V), 1)).astype(f32)
    lg = jax.lax.dot_general(qoh, LT, (((1,), (1,)), ((), ())),
                             preferred_element_type=f32)
    sel = taken > 0.0
    lg_m = jnp.where(sel, lg, NEG)
    m = jnp.max(lg_m, axis=1, keepdims=True)
    e = jnp.where(sel, taken * jnp.exp(lg - m), 0.0)
    w = e / jnp.sum(e, axis=1, keepdims=True)
    out_ref[...] = jnp.dot(w, OT, preferred_element_type=f32) + bo_ref[...]


@jax.jit
def kernel(seq, embed, W1, b1, W2, b2, gamma, beta, Wg, bg, Wq, bq, Wo, bo):
    B, L = seq.shape
    H = HIDDEN_DIM
    V = VOCAB_SIZE
    BB = 4096

    seq_p = seq.astype(jnp.int32)
    row = lambda a: a.reshape(1, -1)
    full = lambda s: pl.BlockSpec(s, lambda i: (0, 0))

    grid = (B // BB,)
    out = pl.pallas_call(
        functools.partial(_fused_kernel, L, BB),
        grid=grid,
        in_specs=[
            pl.BlockSpec((BB, L), lambda i: (i, 0)),
            full((V, H)),
            full((H, 2 * H)), full((1, 2 * H)),
            full((2 * H, H)), full((1, H)),
            full((1, H)), full((1, H)),
            full((H, 1)), full((1, 1)),
            full((H, H)), full((1, H)),
            full((H, V)), full((1, V)),
        ],
        out_specs=pl.BlockSpec((BB, V), lambda i: (i, 0)),
        out_shape=jax.ShapeDtypeStruct((B, V), jnp.float32),
        interpret=INTERPRET,
    )(seq_p, embed, W1, row(b1), W2, row(b2), row(gamma), row(beta),
      Wg.reshape(H, 1), bg.reshape(1, 1), Wq, row(bq), Wo, row(bo))
    return out


# speedup vs baseline: 1.0152x; 1.0152x over previous
import functools

import jax
import jax.numpy as jnp
from jax.experimental import pallas as pl

HIDDEN_DIM = 64
VOCAB_SIZE = 64
MEMORY_SLOTS = 8
NEG = -1e30

INTERPRET = False


def _fused_kernel(L_real, BB, seq_ref, embed_ref, W1_ref, b1_ref, W2_ref,
                  b2_ref, gamma_ref, beta_ref, wg_ref, bg_ref, Wq_ref,
                  bq_ref, Wo_ref, bo_ref, out_ref):
    H = HIDDEN_DIM
    V = VOCAB_SIZE
    f32 = jnp.float32

    E = embed_ref[...]
    p1 = jnp.dot(E, W1_ref[...], preferred_element_type=f32)
    ff1 = jnp.maximum(p1 + b1_ref[...], 0.0)
    ff = jnp.dot(ff1, W2_ref[...], preferred_element_type=f32) + b2_ref[...]
    x = E + ff
    mu = jnp.mean(x, axis=-1, keepdims=True)
    var = jnp.mean((x - mu) ** 2, axis=-1, keepdims=True)
    HT = (x - mu) / jnp.sqrt(var + 1e-5) * gamma_ref[...] + beta_ref[...]

    st = jnp.dot(HT, wg_ref[...], preferred_element_type=f32) + bg_ref[0, 0]
    q_all = jnp.dot(HT, Wq_ref[...], preferred_element_type=f32) + bq_ref[...]
    LT = jax.lax.dot_general(HT, q_all, (((1,), (1,)), ((), ())),
                             preferred_element_type=f32) * (H ** -0.5)
    OT = jnp.dot(HT, Wo_ref[...], preferred_element_type=f32)
    G = (st > st.reshape(1, V)).astype(f32)

    seq = seq_ref[...]
    seq_t = seq.T
    tok = jax.lax.broadcasted_iota(jnp.int32, (V, L_real, BB), 0)
    oh = (seq_t[None, :, :] == tok).astype(f32)
    ones_l = jnp.ones((V, 1, L_real), dtype=f32)
    counts = jax.lax.dot_general(
        ones_l, oh, (((2,), (1,)), ((0,), (0,))),
        preferred_element_type=f32).reshape(V, BB).T

    S = jnp.dot(counts, G, preferred_element_type=f32)
    taken = jnp.minimum(jnp.maximum(8.0 - S, 0.0), counts)

    q_tok = seq[:, L_real - 1][:, None]
    qoh = (q_tok == jax.lax.broadcasted_iota(
        jnp.int32, (BB, V), 1)).astype(f32)
    lg = jax.lax.dot_general(qoh, LT, (((1,), (1,)), ((), ())),
                             preferred_element_type=f32)
    sel = taken > 0.0
    lg_m = jnp.where(sel, lg, NEG)
    m = jnp.max(lg_m, axis=1, keepdims=True)
    e = jnp.where(sel, taken * jnp.exp(lg - m), 0.0)
    w = e / jnp.sum(e, axis=1, keepdims=True)
    out_ref[...] = jnp.dot(w, OT, preferred_element_type=f32) + bo_ref[...]


@jax.jit
def kernel(seq, embed, W1, b1, W2, b2, gamma, beta, Wg, bg, Wq, bq, Wo, bo):
    B, L = seq.shape
    H = HIDDEN_DIM
    V = VOCAB_SIZE
    BB = 2048

    seq_p = seq.astype(jnp.int32)
    row = lambda a: a.reshape(1, -1)
    full = lambda s: pl.BlockSpec(s, lambda i: (0, 0))

    grid = (B // BB,)
    out = pl.pallas_call(
        functools.partial(_fused_kernel, L, BB),
        grid=grid,
        in_specs=[
            pl.BlockSpec((BB, L), lambda i: (i, 0)),
            full((V, H)),
            full((H, 2 * H)), full((1, 2 * H)),
            full((2 * H, H)), full((1, H)),
            full((1, H)), full((1, H)),
            full((H, 1)), full((1, 1)),
            full((H, H)), full((1, H)),
            full((H, V)), full((1, V)),
        ],
        out_specs=pl.BlockSpec((BB, V), lambda i: (i, 0)),
        out_shape=jax.ShapeDtypeStruct((B, V), jnp.float32),
        interpret=INTERPRET,
    )(seq_p, embed, W1, row(b1), W2, row(b2), row(gamma), row(beta),
      Wg.reshape(H, 1), bg.reshape(1, 1), Wq, row(bq), Wo, row(bo))
    return out


# speedup vs baseline: 1.0163x; 1.0012x over previous
import functools

import jax
import jax.numpy as jnp
from jax.experimental import pallas as pl

HIDDEN_DIM = 64
VOCAB_SIZE = 64
MEMORY_SLOTS = 8
NEG = -1e30


def _fused_kernel(L_real, BB, seq_ref, embed_ref, W1_ref, b1_ref, W2_ref,
                  b2_ref, gamma_ref, beta_ref, wg_ref, bg_ref, Wq_ref,
                  bq_ref, Wo_ref, bo_ref, out_ref):
    H = HIDDEN_DIM
    V = VOCAB_SIZE
    f32 = jnp.float32

    E = embed_ref[...]
    p1 = jnp.dot(E, W1_ref[...], preferred_element_type=f32)
    ff1 = jnp.maximum(p1 + b1_ref[...], 0.0)
    ff = jnp.dot(ff1, W2_ref[...], preferred_element_type=f32) + b2_ref[...]
    x = E + ff
    mu = jnp.mean(x, axis=-1, keepdims=True)
    var = jnp.mean((x - mu) ** 2, axis=-1, keepdims=True)
    HT = (x - mu) / jnp.sqrt(var + 1e-5) * gamma_ref[...] + beta_ref[...]

    st = jnp.dot(HT, wg_ref[...], preferred_element_type=f32) + bg_ref[0, 0]
    q_all = jnp.dot(HT, Wq_ref[...], preferred_element_type=f32) + bq_ref[...]
    LT = jax.lax.dot_general(HT, q_all, (((1,), (1,)), ((), ())),
                             preferred_element_type=f32) * (H ** -0.5)
    OT = jnp.dot(HT, Wo_ref[...], preferred_element_type=f32)
    G = (st > st.reshape(1, V)).astype(f32)

    seq = seq_ref[...]
    seq_t = seq.T
    tok = jax.lax.broadcasted_iota(jnp.int32, (V, L_real, BB), 0)
    oh = (seq_t[None, :, :] == tok).astype(f32)
    ones_l = jnp.ones((V, 1, L_real), dtype=f32)
    counts = jax.lax.dot_general(
        ones_l, oh, (((2,), (1,)), ((0,), (0,))),
        preferred_element_type=f32).reshape(V, BB).T

    S = jnp.dot(counts, G, preferred_element_type=f32)
    taken = jnp.minimum(jnp.maximum(8.0 - S, 0.0), counts)

    q_tok = seq[:, L_real - 1][:, None]
    qoh = (q_tok == jax.lax.broadcasted_iota(
        jnp.int32, (BB, V), 1)).astype(f32)
    lg = jax.lax.dot_general(qoh, LT, (((1,), (1,)), ((), ())),
                             preferred_element_type=f32)
    sel = taken > 0.0
    lg_m = jnp.where(sel, lg, NEG)
    m = jnp.max(lg_m, axis=1, keepdims=True)
    e = jnp.where(sel, taken * jnp.exp(lg - m), 0.0)
    w = e / jnp.sum(e, axis=1, keepdims=True)
    out_ref[...] = jnp.dot(w, OT, preferred_element_type=f32) + bo_ref[...]


@jax.jit
def kernel(seq, embed, W1, b1, W2, b2, gamma, beta, Wg, bg, Wq, bq, Wo, bo):
    B, L = seq.shape
    H = HIDDEN_DIM
    V = VOCAB_SIZE
    BB = 2048

    seq_p = seq.astype(jnp.int32)
    row = lambda a: a.reshape(1, -1)
    full = lambda s: pl.BlockSpec(s, lambda i: (0, 0))

    grid = (B // BB,)
    out = pl.pallas_call(
        functools.partial(_fused_kernel, L, BB),
        grid=grid,
        in_specs=[
            pl.BlockSpec((BB, L), lambda i: (i, 0)),
            full((V, H)),
            full((H, 2 * H)), full((1, 2 * H)),
            full((2 * H, H)), full((1, H)),
            full((1, H)), full((1, H)),
            full((H, 1)), full((1, 1)),
            full((H, H)), full((1, H)),
            full((H, V)), full((1, V)),
        ],
        out_specs=pl.BlockSpec((BB, V), lambda i: (i, 0)),
        out_shape=jax.ShapeDtypeStruct((B, V), jnp.float32),
    )(seq_p, embed, W1, row(b1), W2, row(b2), row(gamma), row(beta),
      Wg.reshape(H, 1), bg.reshape(1, 1), Wq, row(bq), Wo, row(bo))
    return out
